# baseline (speedup 1.0000x reference)
import functools

import jax
import jax.numpy as jnp
from jax import lax
from jax.experimental import pallas as pl
from jax.experimental.pallas import tpu as pltpu
from jax.experimental.pallas import tpu_sc as plsc

_K = 10
_REG = 1e-6
_N = 2048
_DIN = 128
_DLAT = 32


_B1 = 256


def _knn_body(xblk_ref, xfull_ref, zblk_ref, out_ref, zp_ref, d_ref):
    xb = xblk_ref[...]
    xf = xfull_ref[...]
    zp_ref[:, : _DLAT] = zblk_ref[...]
    zp_ref[:, _DLAT:] = jnp.zeros((_B1, _DIN - _DLAT), dtype=jnp.float32)
    dot = lax.dot_general(
        xb, xf, (((1,), (1,)), ((), ())), preferred_element_type=jnp.float32
    )
    sqf = jnp.sum(xf * xf, axis=1)
    sqb = jnp.sum(xb * xb, axis=1)
    d_ref[...] = sqb[:, None] + sqf[None, :] - 2.0 * dot

    ci = lax.broadcasted_iota(jnp.int32, (_B1, _N), 1)
    ihalf = lax.broadcasted_iota(jnp.int32, (_B1, _N // 2), 1)
    out = jnp.zeros((16, _B1), dtype=jnp.int32)
    r16 = lax.broadcasted_iota(jnp.int32, (16, _B1), 0)
    for t in range(_K + 1):
        dcur = d_ref[...]
        half = _N // 2
        cc = dcur[:, :half] <= dcur[:, half:]
        v = jnp.where(cc, dcur[:, :half], dcur[:, half:])
        ii = jnp.where(cc, ihalf, ihalf + half)
        while half > 128:
            half //= 2
            cc = v[:, :half] <= v[:, half:]
            v = jnp.where(cc, v[:, :half], v[:, half:])
            ii = jnp.where(cc, ii[:, :half], ii[:, half:])
        m = jnp.min(v, axis=1, keepdims=True)
        cand = jnp.where(v == m, ii, _N * 4)
        idx = jnp.min(cand, axis=1)
        out = jnp.where(r16 == t, idx[None, :], out)
        if t < _K:
            d_ref[...] = jnp.where(ci == idx[:, None], 3.0e38, dcur)
    out_ref[...] = out


def _knn_call(X, Z):
    grid = _N // _B1
    return pl.pallas_call(
        _knn_body,
        grid=(grid,),
        in_specs=[
            pl.BlockSpec((_B1, _DIN), lambda i: (i, 0)),
            pl.BlockSpec((_N, _DIN), lambda i: (0, 0)),
            pl.BlockSpec((_B1, _DLAT), lambda i: (i, 0)),
        ],
        out_specs=[
            pl.BlockSpec((16, _B1), lambda i: (0, i)),
            pl.BlockSpec((_B1, _DIN), lambda i: (i, 0)),
        ],
        out_shape=[
            jax.ShapeDtypeStruct((16, _N), jnp.int32),
            jax.ShapeDtypeStruct((_N, _DIN), jnp.float32),
        ],
        scratch_shapes=[pltpu.VMEM((_B1, _N), jnp.float32)],
    )(X, X, Z)


def _sc_gather(X, Zp, idx):
    b = _N * _K
    nw = 32
    bpw = b // nw
    mesh = plsc.VectorSubcoreMesh(core_axis_name="c", subcore_axis_name="s")

    ch = bpw // 2

    @functools.partial(
        pl.kernel,
        mesh=mesh,
        compiler_params=pltpu.CompilerParams(needs_layout_passes=False),
        out_type=(
            jax.ShapeDtypeStruct((b, _DIN), jnp.float32),
            jax.ShapeDtypeStruct((_DLAT, b), jnp.float32),
        ),
        scratch_types=[
            pltpu.VMEM((bpw,), jnp.int32),
            pltpu.VMEM((ch, _DIN), jnp.float32),
            pltpu.VMEM((ch, _DIN), jnp.float32),
            pltpu.VMEM((_DLAT, bpw), jnp.float32),
            pltpu.SemaphoreType.DMA,
            pltpu.SemaphoreType.DMA,
        ],
    )
    def k(x_hbm, zp_hbm, idx_hbm, xn_hbm, znt_hbm, idx_v, xr_v, zr_v, zt_v, sx, sz):
        wid = lax.axis_index("s") * 2 + lax.axis_index("c")
        base = wid * bpw
        pltpu.sync_copy(idx_hbm.at[pl.ds(_N + base, bpw)], idx_v)
        lanes = lax.iota(jnp.int32, 16)
        for c in range(2):
            sub = idx_v.at[pl.ds(c * ch, ch)]
            cx = pltpu.async_copy(x_hbm.at[sub], xr_v, sx)
            cz = pltpu.async_copy(zp_hbm.at[sub], zr_v, sz)
            cx.wait()
            pltpu.sync_copy(xr_v, xn_hbm.at[pl.ds(base + c * ch, ch)])
            cz.wait()

            def ext(g, _):
                rows = g * 16 + lanes
                for cc in range(_DLAT):
                    v = plsc.load_gather(zr_v, [rows, jnp.full((16,), cc, jnp.int32)])
                    zt_v[cc, pl.ds(c * ch + g * 16, 16)] = v
                return 0

            lax.fori_loop(0, ch // 16, ext, 0, unroll=False)
        pltpu.sync_copy(zt_v, znt_hbm.at[:, pl.ds(base, bpw)])

    return k(X, Zp, idx)


_B3 = 128


def _solve_body(x_ref, z_ref, xn_ref, zn_ref, out_ref):
    i = pl.program_id(0)
    xb = x_ref[...]

    eye = jnp.float32(1.0) * (
        lax.broadcasted_iota(jnp.int32, (_DIN, _DIN), 0)
        == lax.broadcasted_iota(jnp.int32, (_DIN, _DIN), 1)
    )

    def tr(a):
        return lax.dot_general(
            a, eye, (((0,), (0,)), ((), ())), preferred_element_type=jnp.float32
        )

    zbt = tr(z_ref[...])
    xbt = tr(xb)
    diffs = [tr(xn_ref[a]) - xbt for a in range(_K)]

    c = {}
    for a in range(_K):
        for bq in range(a, _K):
            p = jnp.sum(diffs[a] * diffs[bq], axis=0)
            c[(a, bq)] = p + _REG if a == bq else p

    rhs = [jnp.ones((_B3,), dtype=jnp.float32) for _ in range(_K)]

    for kk in range(_K):
        inv = 1.0 / c[(kk, kk)]
        for j in range(kk + 1, _K):
            f = c[(kk, j)] * inv
            for l in range(j, _K):
                c[(j, l)] = c[(j, l)] - f * c[(kk, l)]
            rhs[j] = rhs[j] - f * rhs[kk]

    w = [None] * _K
    for j in range(_K - 1, -1, -1):
        acc = rhs[j]
        for l in range(j + 1, _K):
            acc = acc - c[(j, l)] * w[l]
        w[j] = acc / c[(j, j)]

    s = w[0]
    for j in range(1, _K):
        s = s + w[j]
    sinv = 1.0 / s

    recon = jnp.zeros((_DLAT, _B3), dtype=jnp.float32)
    for kk in range(_K):
        recon = recon + (w[kk] * sinv)[None, :] * zn_ref[:, kk, :]

    partial = jnp.sum((recon - zbt) ** 2)

    @pl.when(i == 0)
    def _():
        out_ref[0, 0] = 0.0

    out_ref[0, 0] += partial


def _solve_call(X, Z, Xn3, Zn3):
    grid = _N // _B3
    return pl.pallas_call(
        _solve_body,
        grid=(grid,),
        in_specs=[
            pl.BlockSpec((_B3, _DIN), lambda i: (i, 0)),
            pl.BlockSpec((_B3, _DLAT), lambda i: (i, 0)),
            pl.BlockSpec((_K, _B3, _DIN), lambda i: (0, i, 0)),
            pl.BlockSpec((_DLAT, _K, _B3), lambda i: (0, 0, i)),
        ],
        out_specs=pl.BlockSpec((1, 1), lambda i: (0, 0), memory_space=pltpu.SMEM),
        out_shape=jax.ShapeDtypeStruct((1, 1), jnp.float32),
    )(X, Z, Xn3, Zn3)


def kernel(X, Z):
    knn, Zp = _knn_call(X, Z)
    idx_flat = jnp.reshape(knn, (-1,))
    Xn, ZnT = _sc_gather(X, Zp, idx_flat)
    Xn3 = jnp.reshape(Xn, (_K, _N, _DIN))
    Zn3 = jnp.reshape(ZnT, (_DLAT, _K, _N))
    ssum = _solve_call(X, Z, Xn3, Zn3)
    return ssum[0, 0] * (1.0 / (_N * _DLAT))


# speedup vs baseline: 1.1016x; 1.1016x over previous
import functools

import jax
import jax.numpy as jnp
from jax import lax
from jax.experimental import pallas as pl
from jax.experimental.pallas import tpu as pltpu
from jax.experimental.pallas import tpu_sc as plsc

_K = 10
_REG = 1e-6
_N = 2048
_DIN = 128
_DLAT = 32


_B1 = 256


def _knn_body(xblk_ref, xfull_ref, zblk_ref, out_ref, zp_ref, d_ref):
    xb = xblk_ref[...]
    xf = xfull_ref[...]
    zp_ref[:, : _DLAT] = zblk_ref[...]
    zp_ref[:, _DLAT:] = jnp.zeros((_B1, _DIN - _DLAT), dtype=jnp.float32)
    dot = lax.dot_general(
        xb, xf, (((1,), (1,)), ((), ())), preferred_element_type=jnp.float32
    )
    sqf = jnp.sum(xf * xf, axis=1)
    sqb = jnp.sum(xb * xb, axis=1)
    d_ref[...] = sqb[:, None] + sqf[None, :] - 2.0 * dot

    ci = lax.broadcasted_iota(jnp.int32, (_B1, _N), 1)
    ihalf = lax.broadcasted_iota(jnp.int32, (_B1, _N // 2), 1)
    out = jnp.zeros((16, _B1), dtype=jnp.int32)
    r16 = lax.broadcasted_iota(jnp.int32, (16, _B1), 0)
    for t in range(_K + 1):
        dcur = d_ref[...]
        half = _N // 2
        cc = dcur[:, :half] <= dcur[:, half:]
        v = jnp.where(cc, dcur[:, :half], dcur[:, half:])
        ii = jnp.where(cc, ihalf, ihalf + half)
        while half > 128:
            half //= 2
            cc = v[:, :half] <= v[:, half:]
            v = jnp.where(cc, v[:, :half], v[:, half:])
            ii = jnp.where(cc, ii[:, :half], ii[:, half:])
        m = jnp.min(v, axis=1, keepdims=True)
        cand = jnp.where(v == m, ii, _N * 4)
        idx = jnp.min(cand, axis=1)
        out = jnp.where(r16 == t, idx[None, :], out)
        if t < _K:
            d_ref[...] = jnp.where(ci == idx[:, None], 3.0e38, dcur)
    out_ref[...] = out


def _knn_call(X, Z):
    grid = _N // _B1
    return pl.pallas_call(
        _knn_body,
        grid=(grid,),
        in_specs=[
            pl.BlockSpec((_B1, _DIN), lambda i: (i, 0)),
            pl.BlockSpec((_N, _DIN), lambda i: (0, 0)),
            pl.BlockSpec((_B1, _DLAT), lambda i: (i, 0)),
        ],
        out_specs=[
            pl.BlockSpec((16, _B1), lambda i: (0, i)),
            pl.BlockSpec((_B1, _DIN), lambda i: (i, 0)),
        ],
        out_shape=[
            jax.ShapeDtypeStruct((16, _N), jnp.int32),
            jax.ShapeDtypeStruct((_N, _DIN), jnp.float32),
        ],
        scratch_shapes=[pltpu.VMEM((_B1, _N), jnp.float32)],
    )(X, X, Z)


def _sc_gather(X, Zp, idx):
    b = _N * _K
    nw = 32
    bpw = b // nw
    mesh = plsc.VectorSubcoreMesh(core_axis_name="c", subcore_axis_name="s")

    ch = bpw // 2

    @functools.partial(
        pl.kernel,
        mesh=mesh,
        out_type=(
            jax.ShapeDtypeStruct((b, _DIN), jnp.float32),
            jax.ShapeDtypeStruct((b, _DIN), jnp.float32),
        ),
        scratch_types=[
            pltpu.VMEM((bpw,), jnp.int32),
            pltpu.VMEM((ch, _DIN), jnp.float32),
            pltpu.VMEM((ch, _DIN), jnp.float32),
            pltpu.SemaphoreType.DMA,
            pltpu.SemaphoreType.DMA,
        ],
    )
    def k(x_hbm, zp_hbm, idx_hbm, xn_hbm, zn_hbm, idx_v, xr_v, zr_v, sx, sz):
        wid = lax.axis_index("s") * 2 + lax.axis_index("c")
        base = wid * bpw
        pltpu.sync_copy(idx_hbm.at[pl.ds(_N + base, bpw)], idx_v)
        for c in range(2):
            sub = idx_v.at[pl.ds(c * ch, ch)]
            cx = pltpu.async_copy(x_hbm.at[sub], xr_v, sx)
            cz = pltpu.async_copy(zp_hbm.at[sub], zr_v, sz)
            cx.wait()
            pltpu.sync_copy(xr_v, xn_hbm.at[pl.ds(base + c * ch, ch)])
            cz.wait()
            pltpu.sync_copy(zr_v, zn_hbm.at[pl.ds(base + c * ch, ch)])

    return k(X, Zp, idx)


_B3 = 128


def _solve_body(x_ref, z_ref, xn_ref, zn_ref, out_ref):
    i = pl.program_id(0)
    xb = x_ref[...]

    eye = jnp.float32(1.0) * (
        lax.broadcasted_iota(jnp.int32, (_DIN, _DIN), 0)
        == lax.broadcasted_iota(jnp.int32, (_DIN, _DIN), 1)
    )

    def tr(a):
        return lax.dot_general(
            a, eye, (((0,), (0,)), ((), ())), preferred_element_type=jnp.float32
        )

    zbt = tr(z_ref[...])
    xbt = tr(xb)
    diffs = [tr(xn_ref[a]) - xbt for a in range(_K)]

    c = {}
    for a in range(_K):
        for bq in range(a, _K):
            p = jnp.sum(diffs[a] * diffs[bq], axis=0)
            c[(a, bq)] = p + _REG if a == bq else p

    rhs = [jnp.ones((_B3,), dtype=jnp.float32) for _ in range(_K)]

    for kk in range(_K):
        inv = 1.0 / c[(kk, kk)]
        for j in range(kk + 1, _K):
            f = c[(kk, j)] * inv
            for l in range(j, _K):
                c[(j, l)] = c[(j, l)] - f * c[(kk, l)]
            rhs[j] = rhs[j] - f * rhs[kk]

    w = [None] * _K
    for j in range(_K - 1, -1, -1):
        acc = rhs[j]
        for l in range(j + 1, _K):
            acc = acc - c[(j, l)] * w[l]
        w[j] = acc / c[(j, j)]

    s = w[0]
    for j in range(1, _K):
        s = s + w[j]
    sinv = 1.0 / s

    recon = jnp.zeros((_DLAT, _B3), dtype=jnp.float32)
    for kk in range(_K):
        recon = recon + (w[kk] * sinv)[None, :] * tr(zn_ref[kk])[: _DLAT, :]

    partial = jnp.sum((recon - zbt) ** 2)

    @pl.when(i == 0)
    def _():
        out_ref[0, 0] = 0.0

    out_ref[0, 0] += partial


def _solve_call(X, Z, Xn3, Zn3):
    grid = _N // _B3
    return pl.pallas_call(
        _solve_body,
        grid=(grid,),
        in_specs=[
            pl.BlockSpec((_B3, _DIN), lambda i: (i, 0)),
            pl.BlockSpec((_B3, _DLAT), lambda i: (i, 0)),
            pl.BlockSpec((_K, _B3, _DIN), lambda i: (0, i, 0)),
            pl.BlockSpec((_K, _B3, _DIN), lambda i: (0, i, 0)),
        ],
        out_specs=pl.BlockSpec((1, 1), lambda i: (0, 0), memory_space=pltpu.SMEM),
        out_shape=jax.ShapeDtypeStruct((1, 1), jnp.float32),
    )(X, Z, Xn3, Zn3)


def kernel(X, Z):
    knn, Zp = _knn_call(X, Z)
    idx_flat = jnp.reshape(knn, (-1,))
    Xn, Zn = _sc_gather(X, Zp, idx_flat)
    Xn3 = jnp.reshape(Xn, (_K, _N, _DIN))
    Zn3 = jnp.reshape(Zn, (_K, _N, _DIN))
    ssum = _solve_call(X, Z, Xn3, Zn3)
    return ssum[0, 0] * (1.0 / (_N * _DLAT))


# speedup vs baseline: 1.2419x; 1.1274x over previous
import functools

import jax
import jax.numpy as jnp
from jax import lax
from jax.experimental import pallas as pl
from jax.experimental.pallas import tpu as pltpu
from jax.experimental.pallas import tpu_sc as plsc

_K = 10
_REG = 1e-6
_N = 2048
_DIN = 128
_DLAT = 32


_B1 = 256


def _knn_body(xblk_ref, xfull_ref, zblk_ref, out_ref, zp_ref, d_ref):
    xb = xblk_ref[...]
    xf = xfull_ref[...]
    zp_ref[:, : _DLAT] = zblk_ref[...]
    zp_ref[:, _DLAT:] = jnp.zeros((_B1, _DIN - _DLAT), dtype=jnp.float32)
    dot = lax.dot_general(
        xb, xf, (((1,), (1,)), ((), ())), preferred_element_type=jnp.float32
    )
    sqf = jnp.sum(xf * xf, axis=1)
    sqb = jnp.sum(xb * xb, axis=1)
    d_ref[...] = sqb[:, None] + sqf[None, :] - 2.0 * dot

    ci = lax.broadcasted_iota(jnp.int32, (_B1, _N), 1)
    ihalf = lax.broadcasted_iota(jnp.int32, (_B1, _N // 2), 1)
    out = jnp.zeros((16, _B1), dtype=jnp.int32)
    r16 = lax.broadcasted_iota(jnp.int32, (16, _B1), 0)
    for t in range(_K + 1):
        dcur = d_ref[...]
        half = _N // 2
        cc = dcur[:, :half] <= dcur[:, half:]
        v = jnp.where(cc, dcur[:, :half], dcur[:, half:])
        ii = jnp.where(cc, ihalf, ihalf + half)
        while half > 128:
            half //= 2
            cc = v[:, :half] <= v[:, half:]
            v = jnp.where(cc, v[:, :half], v[:, half:])
            ii = jnp.where(cc, ii[:, :half], ii[:, half:])
        m = jnp.min(v, axis=1, keepdims=True)
        cand = jnp.where(v == m, ii, _N * 4)
        idx = jnp.min(cand, axis=1)
        out = jnp.where(r16 == t, idx[None, :], out)
        if t < _K:
            d_ref[...] = jnp.where(ci == idx[:, None], 3.0e38, dcur)
    out_ref[...] = out


def _knn_call(X, Z):
    grid = _N // _B1
    return pl.pallas_call(
        _knn_body,
        grid=(grid,),
        in_specs=[
            pl.BlockSpec((_B1, _DIN), lambda i: (i, 0)),
            pl.BlockSpec((_N, _DIN), lambda i: (0, 0)),
            pl.BlockSpec((_B1, _DLAT), lambda i: (i, 0)),
        ],
        out_specs=[
            pl.BlockSpec((16, _B1), lambda i: (0, i)),
            pl.BlockSpec((_B1, _DIN), lambda i: (i, 0)),
        ],
        out_shape=[
            jax.ShapeDtypeStruct((16, _N), jnp.int32),
            jax.ShapeDtypeStruct((_N, _DIN), jnp.float32),
        ],
        scratch_shapes=[pltpu.VMEM((_B1, _N), jnp.float32)],
    )(X, X, Z)


def _sc_gather(X, Zp, idx):
    b = _N * _K
    nw = 32
    bpw = b // nw
    mesh = plsc.VectorSubcoreMesh(core_axis_name="c", subcore_axis_name="s")

    ch = bpw // 2

    @functools.partial(
        pl.kernel,
        mesh=mesh,
        out_type=(
            jax.ShapeDtypeStruct((b, _DIN), jnp.float32),
            jax.ShapeDtypeStruct((b, _DIN), jnp.float32),
        ),
        scratch_types=[
            pltpu.VMEM((bpw,), jnp.int32),
            pltpu.VMEM((ch, _DIN), jnp.float32),
            pltpu.VMEM((ch, _DIN), jnp.float32),
            pltpu.VMEM_SHARED((_N, _DIN), jnp.float32),
            pltpu.VMEM_SHARED((_N, _DIN), jnp.float32),
            pltpu.SemaphoreType.DMA,
            pltpu.SemaphoreType.DMA,
        ],
    )
    def k(x_hbm, zp_hbm, idx_hbm, xn_hbm, zn_hbm, idx_v, xr_v, zr_v, xs, zs, sx, sz):
        wid = lax.axis_index("s") * 2 + lax.axis_index("c")
        base = wid * bpw
        @pl.when(lax.axis_index("s") == 0)
        def _():
            pltpu.sync_copy(x_hbm, xs)
            pltpu.sync_copy(zp_hbm, zs)

        pltpu.sync_copy(idx_hbm.at[pl.ds(_N + base, bpw)], idx_v)
        plsc.subcore_barrier()
        for c in range(2):
            sub = idx_v.at[pl.ds(c * ch, ch)]
            cx = pltpu.async_copy(xs.at[sub], xr_v, sx)
            cz = pltpu.async_copy(zs.at[sub], zr_v, sz)
            cx.wait()
            pltpu.sync_copy(xr_v, xn_hbm.at[pl.ds(base + c * ch, ch)])
            cz.wait()
            pltpu.sync_copy(zr_v, zn_hbm.at[pl.ds(base + c * ch, ch)])

    return k(X, Zp, idx)


_B3 = 128


def _solve_body(x_ref, z_ref, xn_ref, zn_ref, out_ref):
    i = pl.program_id(0)
    xb = x_ref[...]

    eye = jnp.float32(1.0) * (
        lax.broadcasted_iota(jnp.int32, (_DIN, _DIN), 0)
        == lax.broadcasted_iota(jnp.int32, (_DIN, _DIN), 1)
    )

    def tr(a):
        return lax.dot_general(
            a, eye, (((0,), (0,)), ((), ())), preferred_element_type=jnp.float32
        )

    zbt = tr(z_ref[...])
    xbt = tr(xb)
    diffs = [tr(xn_ref[a]) - xbt for a in range(_K)]

    c = {}
    for a in range(_K):
        for bq in range(a, _K):
            p = jnp.sum(diffs[a] * diffs[bq], axis=0)
            c[(a, bq)] = p + _REG if a == bq else p

    rhs = [jnp.ones((_B3,), dtype=jnp.float32) for _ in range(_K)]

    for kk in range(_K):
        inv = 1.0 / c[(kk, kk)]
        for j in range(kk + 1, _K):
            f = c[(kk, j)] * inv
            for l in range(j, _K):
                c[(j, l)] = c[(j, l)] - f * c[(kk, l)]
            rhs[j] = rhs[j] - f * rhs[kk]

    w = [None] * _K
    for j in range(_K - 1, -1, -1):
        acc = rhs[j]
        for l in range(j + 1, _K):
            acc = acc - c[(j, l)] * w[l]
        w[j] = acc / c[(j, j)]

    s = w[0]
    for j in range(1, _K):
        s = s + w[j]
    sinv = 1.0 / s

    recon = jnp.zeros((_DLAT, _B3), dtype=jnp.float32)
    for kk in range(_K):
        recon = recon + (w[kk] * sinv)[None, :] * tr(zn_ref[kk])[: _DLAT, :]

    partial = jnp.sum((recon - zbt) ** 2)

    @pl.when(i == 0)
    def _():
        out_ref[0, 0] = 0.0

    out_ref[0, 0] += partial


def _solve_call(X, Z, Xn3, Zn3):
    grid = _N // _B3
    return pl.pallas_call(
        _solve_body,
        grid=(grid,),
        in_specs=[
            pl.BlockSpec((_B3, _DIN), lambda i: (i, 0)),
            pl.BlockSpec((_B3, _DLAT), lambda i: (i, 0)),
            pl.BlockSpec((_K, _B3, _DIN), lambda i: (0, i, 0)),
            pl.BlockSpec((_K, _B3, _DIN), lambda i: (0, i, 0)),
        ],
        out_specs=pl.BlockSpec((1, 1), lambda i: (0, 0), memory_space=pltpu.SMEM),
        out_shape=jax.ShapeDtypeStruct((1, 1), jnp.float32),
    )(X, Z, Xn3, Zn3)


def kernel(X, Z):
    knn, Zp = _knn_call(X, Z)
    idx_flat = jnp.reshape(knn, (-1,))
    Xn, Zn = _sc_gather(X, Zp, idx_flat)
    Xn3 = jnp.reshape(Xn, (_K, _N, _DIN))
    Zn3 = jnp.reshape(Zn, (_K, _N, _DIN))
    ssum = _solve_call(X, Z, Xn3, Zn3)
    return ssum[0, 0] * (1.0 / (_N * _DLAT))


# speedup vs baseline: 1.3066x; 1.0521x over previous
import functools

import jax
import jax.numpy as jnp
from jax import lax
from jax.experimental import pallas as pl
from jax.experimental.pallas import tpu as pltpu
from jax.experimental.pallas import tpu_sc as plsc

_K = 10
_REG = 1e-6
_N = 2048
_DIN = 128
_DLAT = 32


_B1 = 512


def _knn_body(xblk_ref, xfull_ref, zblk_ref, out_ref, zp_ref, d_ref):
    xb = xblk_ref[...]
    xf = xfull_ref[...]
    zp_ref[:, : _DLAT] = zblk_ref[...]
    zp_ref[:, _DLAT:] = jnp.zeros((_B1, _DIN - _DLAT), dtype=jnp.float32)
    dot = lax.dot_general(
        xb, xf, (((1,), (1,)), ((), ())), preferred_element_type=jnp.float32
    )
    sqf = jnp.sum(xf * xf, axis=1)
    sqb = jnp.sum(xb * xb, axis=1)
    d_ref[...] = sqb[:, None] + sqf[None, :] - 2.0 * dot

    ci = lax.broadcasted_iota(jnp.int32, (_B1, _N), 1)
    ihalf = lax.broadcasted_iota(jnp.int32, (_B1, _N // 2), 1)
    out = jnp.zeros((16, _B1), dtype=jnp.int32)
    r16 = lax.broadcasted_iota(jnp.int32, (16, _B1), 0)
    for t in range(_K + 1):
        dcur = d_ref[...]
        half = _N // 2
        cc = dcur[:, :half] <= dcur[:, half:]
        v = jnp.where(cc, dcur[:, :half], dcur[:, half:])
        ii = jnp.where(cc, ihalf, ihalf + half)
        while half > 128:
            half //= 2
            cc = v[:, :half] <= v[:, half:]
            v = jnp.where(cc, v[:, :half], v[:, half:])
            ii = jnp.where(cc, ii[:, :half], ii[:, half:])
        m = jnp.min(v, axis=1, keepdims=True)
        cand = jnp.where(v == m, ii, _N * 4)
        idx = jnp.min(cand, axis=1)
        out = jnp.where(r16 == t, idx[None, :], out)
        if t < _K:
            d_ref[...] = jnp.where(ci == idx[:, None], 3.0e38, dcur)
    out_ref[...] = out


def _knn_call(X, Z):
    grid = _N // _B1
    return pl.pallas_call(
        _knn_body,
        grid=(grid,),
        in_specs=[
            pl.BlockSpec((_B1, _DIN), lambda i: (i, 0)),
            pl.BlockSpec((_N, _DIN), lambda i: (0, 0)),
            pl.BlockSpec((_B1, _DLAT), lambda i: (i, 0)),
        ],
        out_specs=[
            pl.BlockSpec((16, _B1), lambda i: (0, i)),
            pl.BlockSpec((_B1, _DIN), lambda i: (i, 0)),
        ],
        out_shape=[
            jax.ShapeDtypeStruct((16, _N), jnp.int32),
            jax.ShapeDtypeStruct((_N, _DIN), jnp.float32),
        ],
        scratch_shapes=[pltpu.VMEM((_B1, _N), jnp.float32)],
    )(X, X, Z)


def _sc_gather(X, Zp, idx):
    b = _N * _K
    nw = 32
    bpw = b // nw
    mesh = plsc.VectorSubcoreMesh(core_axis_name="c", subcore_axis_name="s")

    ch = bpw // 2

    @functools.partial(
        pl.kernel,
        mesh=mesh,
        out_type=(
            jax.ShapeDtypeStruct((b, _DIN), jnp.float32),
            jax.ShapeDtypeStruct((b, _DIN), jnp.float32),
        ),
        scratch_types=[
            pltpu.VMEM((bpw,), jnp.int32),
            pltpu.VMEM((ch, _DIN), jnp.float32),
            pltpu.VMEM((ch, _DIN), jnp.float32),
            pltpu.VMEM_SHARED((_N, _DIN), jnp.float32),
            pltpu.VMEM_SHARED((_N, _DIN), jnp.float32),
            pltpu.SemaphoreType.DMA,
            pltpu.SemaphoreType.DMA,
        ],
    )
    def k(x_hbm, zp_hbm, idx_hbm, xn_hbm, zn_hbm, idx_v, xr_v, zr_v, xs, zs, sx, sz):
        wid = lax.axis_index("s") * 2 + lax.axis_index("c")
        base = wid * bpw
        @pl.when(lax.axis_index("s") == 0)
        def _():
            pltpu.sync_copy(x_hbm, xs)
            pltpu.sync_copy(zp_hbm, zs)

        pltpu.sync_copy(idx_hbm.at[pl.ds(_N + base, bpw)], idx_v)
        plsc.subcore_barrier()
        for c in range(2):
            sub = idx_v.at[pl.ds(c * ch, ch)]
            cx = pltpu.async_copy(xs.at[sub], xr_v, sx)
            cz = pltpu.async_copy(zs.at[sub], zr_v, sz)
            cx.wait()
            pltpu.sync_copy(xr_v, xn_hbm.at[pl.ds(base + c * ch, ch)])
            cz.wait()
            pltpu.sync_copy(zr_v, zn_hbm.at[pl.ds(base + c * ch, ch)])

    return k(X, Zp, idx)


_B3 = 128


def _solve_body(x_ref, z_ref, xn_ref, zn_ref, out_ref):
    i = pl.program_id(0)
    xb = x_ref[...]

    eye = jnp.float32(1.0) * (
        lax.broadcasted_iota(jnp.int32, (_DIN, _DIN), 0)
        == lax.broadcasted_iota(jnp.int32, (_DIN, _DIN), 1)
    )

    def tr(a):
        return lax.dot_general(
            a, eye, (((0,), (0,)), ((), ())), preferred_element_type=jnp.float32
        )

    zbt = tr(z_ref[...])
    xbt = tr(xb)
    diffs = [tr(xn_ref[a]) - xbt for a in range(_K)]

    c = {}
    for a in range(_K):
        for bq in range(a, _K):
            p = jnp.sum(diffs[a] * diffs[bq], axis=0)
            c[(a, bq)] = p + _REG if a == bq else p

    rhs = [jnp.ones((_B3,), dtype=jnp.float32) for _ in range(_K)]

    for kk in range(_K):
        inv = 1.0 / c[(kk, kk)]
        for j in range(kk + 1, _K):
            f = c[(kk, j)] * inv
            for l in range(j, _K):
                c[(j, l)] = c[(j, l)] - f * c[(kk, l)]
            rhs[j] = rhs[j] - f * rhs[kk]

    w = [None] * _K
    for j in range(_K - 1, -1, -1):
        acc = rhs[j]
        for l in range(j + 1, _K):
            acc = acc - c[(j, l)] * w[l]
        w[j] = acc / c[(j, j)]

    s = w[0]
    for j in range(1, _K):
        s = s + w[j]
    sinv = 1.0 / s

    recon = jnp.zeros((_DLAT, _B3), dtype=jnp.float32)
    for kk in range(_K):
        recon = recon + (w[kk] * sinv)[None, :] * tr(zn_ref[kk])[: _DLAT, :]

    partial = jnp.sum((recon - zbt) ** 2)

    @pl.when(i == 0)
    def _():
        out_ref[0, 0] = 0.0

    out_ref[0, 0] += partial


def _solve_call(X, Z, Xn3, Zn3):
    grid = _N // _B3
    return pl.pallas_call(
        _solve_body,
        grid=(grid,),
        in_specs=[
            pl.BlockSpec((_B3, _DIN), lambda i: (i, 0)),
            pl.BlockSpec((_B3, _DLAT), lambda i: (i, 0)),
            pl.BlockSpec((_K, _B3, _DIN), lambda i: (0, i, 0)),
            pl.BlockSpec((_K, _B3, _DIN), lambda i: (0, i, 0)),
        ],
        out_specs=pl.BlockSpec((1, 1), lambda i: (0, 0), memory_space=pltpu.SMEM),
        out_shape=jax.ShapeDtypeStruct((1, 1), jnp.float32),
    )(X, Z, Xn3, Zn3)


def kernel(X, Z):
    knn, Zp = _knn_call(X, Z)
    idx_flat = jnp.reshape(knn, (-1,))
    Xn, Zn = _sc_gather(X, Zp, idx_flat)
    Xn3 = jnp.reshape(Xn, (_K, _N, _DIN))
    Zn3 = jnp.reshape(Zn, (_K, _N, _DIN))
    ssum = _solve_call(X, Z, Xn3, Zn3)
    return ssum[0, 0] * (1.0 / (_N * _DLAT))


# speedup vs baseline: 1.3635x; 1.0436x over previous
import functools

import jax
import jax.numpy as jnp
from jax import lax
from jax.experimental import pallas as pl
from jax.experimental.pallas import tpu as pltpu
from jax.experimental.pallas import tpu_sc as plsc

_K = 10
_REG = 1e-6
_N = 2048
_DIN = 128
_DLAT = 32


_B1 = 512


def _knn_body(xblk_ref, xfull_ref, zblk_ref, out_ref, zp_ref, d_ref):
    xb = xblk_ref[...]
    xf = xfull_ref[...]
    zp_ref[:, : _DLAT] = zblk_ref[...]
    zp_ref[:, _DLAT:] = jnp.zeros((_B1, _DIN - _DLAT), dtype=jnp.float32)
    dot = lax.dot_general(
        xb, xf, (((1,), (1,)), ((), ())), preferred_element_type=jnp.float32
    )
    sqf = jnp.sum(xf * xf, axis=1)
    sqb = jnp.sum(xb * xb, axis=1)
    d_ref[...] = sqb[:, None] + sqf[None, :] - 2.0 * dot

    ci = lax.broadcasted_iota(jnp.int32, (_B1, _N), 1)
    ihalf = lax.broadcasted_iota(jnp.int32, (_B1, _N // 2), 1)
    out = jnp.zeros((16, _B1), dtype=jnp.int32)
    r16 = lax.broadcasted_iota(jnp.int32, (16, _B1), 0)
    for t in range(_K + 1):
        dcur = d_ref[...]
        half = _N // 2
        cc = dcur[:, :half] <= dcur[:, half:]
        v = jnp.where(cc, dcur[:, :half], dcur[:, half:])
        ii = jnp.where(cc, ihalf, ihalf + half)
        while half > 128:
            half //= 2
            cc = v[:, :half] <= v[:, half:]
            v = jnp.where(cc, v[:, :half], v[:, half:])
            ii = jnp.where(cc, ii[:, :half], ii[:, half:])
        m = jnp.min(v, axis=1, keepdims=True)
        cand = jnp.where(v == m, ii, _N * 4)
        idx = jnp.min(cand, axis=1)
        out = jnp.where(r16 == t, idx[None, :], out)
        if t < _K:
            d_ref[...] = jnp.where(ci == idx[:, None], 3.0e38, dcur)
    out_ref[...] = out


def _knn_call(X, Z):
    grid = _N // _B1
    return pl.pallas_call(
        _knn_body,
        grid=(grid,),
        in_specs=[
            pl.BlockSpec((_B1, _DIN), lambda i: (i, 0)),
            pl.BlockSpec((_N, _DIN), lambda i: (0, 0)),
            pl.BlockSpec((_B1, _DLAT), lambda i: (i, 0)),
        ],
        out_specs=[
            pl.BlockSpec((16, _B1), lambda i: (0, i)),
            pl.BlockSpec((_B1, _DIN), lambda i: (i, 0)),
        ],
        out_shape=[
            jax.ShapeDtypeStruct((16, _N), jnp.int32),
            jax.ShapeDtypeStruct((_N, _DIN), jnp.float32),
        ],
        scratch_shapes=[pltpu.VMEM((_B1, _N), jnp.float32)],
    )(X, X, Z)


def _sc_gather(X, Zp, idx):
    b = _N * _K
    nw = 32
    bpw = b // nw
    mesh = plsc.VectorSubcoreMesh(core_axis_name="c", subcore_axis_name="s")

    ch = bpw // 2

    @functools.partial(
        pl.kernel,
        mesh=mesh,
        out_type=(
            jax.ShapeDtypeStruct((b, _DIN), jnp.float32),
            jax.ShapeDtypeStruct((b, _DIN), jnp.float32),
        ),
        scratch_types=[
            pltpu.VMEM((bpw,), jnp.int32),
            pltpu.VMEM((ch, _DIN), jnp.float32),
            pltpu.VMEM((ch, _DIN), jnp.float32),
            pltpu.VMEM_SHARED((_N, _DIN), jnp.float32),
            pltpu.VMEM_SHARED((_N, _DIN), jnp.float32),
            pltpu.SemaphoreType.DMA,
            pltpu.SemaphoreType.DMA,
        ],
    )
    def k(x_hbm, zp_hbm, idx_hbm, xn_hbm, zn_hbm, idx_v, xr_v, zr_v, xs, zs, sx, sz):
        wid = lax.axis_index("s") * 2 + lax.axis_index("c")
        base = wid * bpw
        @pl.when(lax.axis_index("s") == 0)
        def _():
            pltpu.sync_copy(x_hbm, xs)
            pltpu.sync_copy(zp_hbm, zs)

        pltpu.sync_copy(idx_hbm.at[pl.ds(_N + base, bpw)], idx_v)
        plsc.subcore_barrier()
        for c in range(2):
            sub = idx_v.at[pl.ds(c * ch, ch)]
            cx = pltpu.async_copy(xs.at[sub], xr_v, sx)
            cz = pltpu.async_copy(zs.at[sub], zr_v, sz)
            cx.wait()
            pltpu.sync_copy(xr_v, xn_hbm.at[pl.ds(base + c * ch, ch)])
            cz.wait()
            pltpu.sync_copy(zr_v, zn_hbm.at[pl.ds(base + c * ch, ch)])

    return k(X, Zp, idx)


_B3 = 256


def _solve_body(x_ref, z_ref, xn_ref, zn_ref, out_ref):
    i = pl.program_id(0)
    xb = x_ref[...]

    eye = jnp.float32(1.0) * (
        lax.broadcasted_iota(jnp.int32, (_B3, _B3), 0)
        == lax.broadcasted_iota(jnp.int32, (_B3, _B3), 1)
    )

    def tr(a):
        return lax.dot_general(
            a, eye, (((0,), (0,)), ((), ())), preferred_element_type=jnp.float32
        )

    zbt = tr(z_ref[...])
    xbt = tr(xb)
    diffs = [tr(xn_ref[a]) - xbt for a in range(_K)]

    c = {}
    for a in range(_K):
        for bq in range(a, _K):
            p = jnp.sum(diffs[a] * diffs[bq], axis=0)
            c[(a, bq)] = p + _REG if a == bq else p

    rhs = [jnp.ones((_B3,), dtype=jnp.float32) for _ in range(_K)]

    for kk in range(_K):
        inv = 1.0 / c[(kk, kk)]
        for j in range(kk + 1, _K):
            f = c[(kk, j)] * inv
            for l in range(j, _K):
                c[(j, l)] = c[(j, l)] - f * c[(kk, l)]
            rhs[j] = rhs[j] - f * rhs[kk]

    w = [None] * _K
    for j in range(_K - 1, -1, -1):
        acc = rhs[j]
        for l in range(j + 1, _K):
            acc = acc - c[(j, l)] * w[l]
        w[j] = acc / c[(j, j)]

    s = w[0]
    for j in range(1, _K):
        s = s + w[j]
    sinv = 1.0 / s

    recon = jnp.zeros((_DLAT, _B3), dtype=jnp.float32)
    for kk in range(_K):
        recon = recon + (w[kk] * sinv)[None, :] * tr(zn_ref[kk])[: _DLAT, :]

    partial = jnp.sum((recon - zbt) ** 2)

    @pl.when(i == 0)
    def _():
        out_ref[0, 0] = 0.0

    out_ref[0, 0] += partial


def _solve_call(X, Z, Xn3, Zn3):
    grid = _N // _B3
    return pl.pallas_call(
        _solve_body,
        grid=(grid,),
        in_specs=[
            pl.BlockSpec((_B3, _DIN), lambda i: (i, 0)),
            pl.BlockSpec((_B3, _DLAT), lambda i: (i, 0)),
            pl.BlockSpec((_K, _B3, _DIN), lambda i: (0, i, 0)),
            pl.BlockSpec((_K, _B3, _DIN), lambda i: (0, i, 0)),
        ],
        out_specs=pl.BlockSpec((1, 1), lambda i: (0, 0), memory_space=pltpu.SMEM),
        out_shape=jax.ShapeDtypeStruct((1, 1), jnp.float32),
    )(X, Z, Xn3, Zn3)


def kernel(X, Z):
    knn, Zp = _knn_call(X, Z)
    idx_flat = jnp.reshape(knn, (-1,))
    Xn, Zn = _sc_gather(X, Zp, idx_flat)
    Xn3 = jnp.reshape(Xn, (_K, _N, _DIN))
    Zn3 = jnp.reshape(Zn, (_K, _N, _DIN))
    ssum = _solve_call(X, Z, Xn3, Zn3)
    return ssum[0, 0] * (1.0 / (_N * _DLAT))
